# baseline (speedup 1.0000x reference)
import jax
import jax.numpy as jnp
from jax import lax
from jax.experimental import pallas as pl
from jax.experimental.pallas import tpu as pltpu

_Q = 4
_K = 1024
_D = 128
_BN = 1024
_H = _BN // 2


def _rvq_kernel(x_ref, cb_ref, xq_ref, loss_ref,
                i0_ref, i1_ref, i2_ref, i3_ref,
                res_ref, c2_ref, cbd_ref, cbg_ref):
    idx_refs = (i0_ref, i1_ref, i2_ref, i3_ref)

    @pl.when(pl.program_id(0) == 0)
    def _init_codebook_scratch():
        riota = lax.broadcasted_iota(jnp.int32, (_K, 1), 0).astype(jnp.float32)
        ihi = riota.astype(jnp.bfloat16)
        ilo = (riota - ihi.astype(jnp.float32)).astype(jnp.bfloat16)
        aux = jnp.concatenate(
            [ihi, ilo, jnp.ones((_K, 1), jnp.bfloat16),
             jnp.zeros((_K, _D - 3), jnp.bfloat16)], axis=1)
        for q in range(_Q):
            cb = cb_ref[q]
            c2_ref[q] = jnp.sum(cb * cb, axis=1)[None, :]
            hi = cb.astype(jnp.bfloat16)
            lo = (cb - hi.astype(jnp.float32)).astype(jnp.bfloat16)
            cbd_ref[q] = hi
            cbg_ref[q] = jnp.concatenate([hi, lo, aux], axis=1)

    sse = jnp.zeros((1, 1), jnp.float32)
    for q in range(_Q):
        for h in range(2):
            hs = pl.ds(h * _H, _H)
            res = x_ref[hs, :] if q == 0 else res_ref[hs, :]
            z2 = jnp.sum(res * res, axis=1, keepdims=True)
            nres = (res * -2.0).astype(jnp.bfloat16)
            s = lax.dot_general(nres, cbd_ref[q], (((1,), (1,)), ((), ())),
                                preferred_element_type=jnp.float32)
            d = (z2 + s) + c2_ref[q]
            m = jnp.min(d, axis=1, keepdims=True)
            mask = d == m
            g = lax.dot_general(mask.astype(jnp.bfloat16), cbg_ref[q],
                                (((1,), (0,)), ((), ())),
                                preferred_element_type=jnp.float32)
            zq = g[:, 0:_D] + g[:, _D:2 * _D]
            idxf = g[:, 2 * _D] + g[:, 2 * _D + 1]
            cnt = g[:, 2 * _D + 2]
            res_ref[hs, :] = res - zq
            idx_refs[q][hs, :] = idxf.astype(jnp.int32)[:, None]
            if q == 0:
                xq_ref[hs, :] = zq
            else:
                xq_ref[hs, :] = xq_ref[hs, :] + zq

            @pl.when(jnp.max(cnt) > 1.0)
            def _fix(res=res, mask=mask, zq=zq, q=q, hs=hs):
                liota = lax.broadcasted_iota(
                    jnp.int32, (_H, _K), 1).astype(jnp.float32)
                idxf2 = jnp.min(jnp.where(mask, liota, 2048.0), axis=1)
                onehot = (liota == idxf2[:, None]).astype(jnp.bfloat16)
                g2 = lax.dot_general(onehot, cbg_ref[q],
                                     (((1,), (0,)), ((), ())),
                                     preferred_element_type=jnp.float32)
                zqx = g2[:, 0:_D] + g2[:, _D:2 * _D]
                res_ref[hs, :] = res - zqx
                idx_refs[q][hs, :] = idxf2.astype(jnp.int32)[:, None]
                xq_ref[hs, :] = (xq_ref[hs, :] - zq) + zqx

            rr = res_ref[hs, :]
            sse = sse + jnp.sum(rr * rr, keepdims=True).reshape(1, 1)
    loss_ref[...] = jnp.broadcast_to(sse[None], (1, 1, 128))


def kernel(x, codebooks):
    n = x.shape[0]
    nblk = n // _BN
    out_shape = (
        jax.ShapeDtypeStruct((n, _D), jnp.float32),
        jax.ShapeDtypeStruct((nblk, 1, 128), jnp.float32),
        jax.ShapeDtypeStruct((n, 1), jnp.int32),
        jax.ShapeDtypeStruct((n, 1), jnp.int32),
        jax.ShapeDtypeStruct((n, 1), jnp.int32),
        jax.ShapeDtypeStruct((n, 1), jnp.int32),
    )
    xq, losses, i0, i1, i2, i3 = pl.pallas_call(
        _rvq_kernel,
        grid=(nblk,),
        in_specs=[
            pl.BlockSpec((_BN, _D), lambda i: (i, 0)),
            pl.BlockSpec((_Q, _K, _D), lambda i: (0, 0, 0)),
        ],
        out_specs=[
            pl.BlockSpec((_BN, _D), lambda i: (i, 0)),
            pl.BlockSpec((1, 1, 128), lambda i: (i, 0, 0)),
            pl.BlockSpec((_BN, 1), lambda i: (i, 0)),
            pl.BlockSpec((_BN, 1), lambda i: (i, 0)),
            pl.BlockSpec((_BN, 1), lambda i: (i, 0)),
            pl.BlockSpec((_BN, 1), lambda i: (i, 0)),
        ],
        out_shape=out_shape,
        scratch_shapes=[
            pltpu.VMEM((_BN, _D), jnp.float32),
            pltpu.VMEM((_Q, 1, _K), jnp.float32),
            pltpu.VMEM((_Q, _K, _D), jnp.bfloat16),
            pltpu.VMEM((_Q, _K, 3 * _D), jnp.bfloat16),
        ],
        compiler_params=pltpu.CompilerParams(
            dimension_semantics=("arbitrary",)),
    )(x, codebooks)
    indices = jnp.concatenate([i0, i1, i2, i3], axis=1)
    loss = jnp.sum(losses[:, 0, 0]) * (1.25 / (_Q * n * _D))
    return xq, loss, indices
